# Initial kernel scaffold; baseline (speedup 1.0000x reference)
import functools

import jax
import jax.numpy as jnp
from jax import lax
from jax.experimental import pallas as pl
from jax.experimental.pallas import tpu as pltpu
from jax.experimental.pallas import tpu_sc as plsc

_N, _K, _D, _P = 10000, 16, 128, 16
_H = _D * 4
_HP = _P * 2
_BN = 200


def _make_sc_gather(n_idx, d, chunk):
  info = plsc.get_sparse_core_info()
  nw = info.num_cores * info.num_subcores
  b_per_w = n_idx // nw
  assert b_per_w * nw == n_idx and b_per_w % chunk == 0 and chunk % 8 == 0
  nchunks = b_per_w // chunk
  mesh = plsc.VectorSubcoreMesh(core_axis_name="c", subcore_axis_name="s")

  @functools.partial(
      pl.kernel,
      out_type=jax.ShapeDtypeStruct((n_idx, d), jnp.float32),
      mesh=mesh,
      scratch_types=[
          pltpu.VMEM((chunk,), jnp.int32),
          pltpu.VMEM((chunk, d), jnp.float32),
          pltpu.SemaphoreType.DMA,
      ],
  )
  def gather_kernel(table_hbm, idx_hbm, out_hbm, idx_v, rows_v, sem):
    wid = lax.axis_index("s") * info.num_cores + lax.axis_index("c")
    base = wid * b_per_w

    @pl.loop(0, nchunks)
    def _chunk(ci):
      off = pl.multiple_of(base + ci * chunk, 8)
      pltpu.sync_copy(idx_hbm.at[pl.ds(off, chunk)], idx_v)
      pltpu.async_copy(table_hbm.at[idx_v], rows_v, sem).wait()
      pltpu.sync_copy(rows_v, out_hbm.at[pl.ds(off, chunk)])

  return gather_kernel


_gather128 = _make_sc_gather(_N * _K, _D, 200)
_gather32 = _make_sc_gather(_N * _K, _HP, 1000)


def _ln(x, g, b, eps=1e-5):
  mu = jnp.mean(x, axis=-1, keepdims=True)
  var = jnp.mean((x - mu) ** 2, axis=-1, keepdims=True)
  return (x - mu) / jnp.sqrt(var + eps) * g + b


def _tc1_body(local_ref, lg_ref, paire_ref, pme_ref,
              w1s_ref, w1n_ref, w1p_ref, w1b_ref,
              wgl_ref, bgl_ref, g1_ref, b1_ref,
              wg_ref, wv_ref, wo_ref, g2_ref, b2_ref,
              w2s_ref, w2n_ref,
              out_local_ref, out_a2_ref, out_b2_ref):
  f32 = jnp.float32
  local = local_ref[...]
  a1 = jnp.dot(local, w1s_ref[...], preferred_element_type=f32)
  h = jnp.dot(lg_ref[...], w1n_ref[...], preferred_element_type=f32)
  h = h + jnp.dot(paire_ref[...], w1p_ref[...], preferred_element_type=f32)
  a1e = jnp.broadcast_to(a1[:, None, :], (_BN, _K, _H)).reshape(_BN * _K, _H)
  h = jnp.maximum(h + a1e, 0.0) * pme_ref[...]
  s = h.reshape(_BN, _K, _H).sum(axis=1)
  lu = jnp.dot(s, w1b_ref[...], preferred_element_type=f32) * (1.0 / _K)
  gate = jax.nn.sigmoid(
      jnp.dot(local, wgl_ref[...], preferred_element_type=f32) + bgl_ref[...])
  x = _ln(local + lu * gate, g1_ref[...], b1_ref[...])
  g = jnp.dot(x, wg_ref[...], preferred_element_type=f32)
  v = jnp.dot(x, wv_ref[...], preferred_element_type=f32)
  y = jnp.dot(jax.nn.silu(g) * v, wo_ref[...], preferred_element_type=f32)
  x2 = _ln(x + y, g2_ref[...], b2_ref[...])
  out_local_ref[...] = x2
  out_a2_ref[...] = jnp.dot(x2, w2s_ref[...], preferred_element_type=f32)
  out_b2_ref[...] = jnp.dot(x2, w2n_ref[...], preferred_element_type=f32)


def _tc2_body(a2_ref, b2g_ref, paire_ref,
              w2p_ref, w2b_ref, wgp_ref, bgp_ref, g3_ref, b3_ref,
              out_ref):
  f32 = jnp.float32
  a2 = a2_ref[...]
  a2e = jnp.broadcast_to(a2[:, None, :], (_BN, _K, _HP)).reshape(_BN * _K, _HP)
  paire = paire_ref[...]
  h = jnp.dot(paire, w2p_ref[...], preferred_element_type=f32)
  h = jnp.maximum(h + b2g_ref[...] + a2e, 0.0)
  pu = jnp.dot(h, w2b_ref[...], preferred_element_type=f32)
  gate = jax.nn.sigmoid(
      jnp.dot(paire, wgp_ref[...], preferred_element_type=f32) + bgp_ref[...])
  out_ref[...] = _ln(paire + pu * gate, g3_ref[...], b3_ref[...])


def _node_spec(d):
  return pl.BlockSpec((_BN, d), lambda i: (i, 0))


def _edge_spec(d):
  return pl.BlockSpec((_BN * _K, d), lambda i: (i, 0))


def _w_spec(r, c):
  return pl.BlockSpec((r, c), lambda i: (0, 0))


def kernel(local, pair, neighbours, mask, W1a, W1b, Wgate_l, bgate_l, g1, b1,
           Wg, Wv, Wo, g2, b2, W2a, W2b, Wgate_p, bgate_p, g3, b3):
  n, k = neighbours.shape
  d, p, h, hp = _D, _P, _H, _HP

  idx = jnp.where(neighbours < 0, 0, neighbours).reshape(n * k)
  pme = (mask[:, None] * (neighbours != -1).astype(jnp.float32)).reshape(n * k, 1)
  paire = pair.reshape(n * k, p)

  w1s, w1n, w1p = W1a[:d], W1a[d:2 * d], W1a[2 * d:]
  w2s, w2n, w2p = W2a[:d], W2a[d:2 * d], W2a[2 * d:]
  row = lambda a: a.reshape(1, -1)

  lg = _gather128(local, idx)

  grid = (n // _BN,)
  local2, a2, b2t = pl.pallas_call(
      _tc1_body,
      grid=grid,
      in_specs=[
          _node_spec(d), _edge_spec(d), _edge_spec(p), _edge_spec(1),
          _w_spec(d, h), _w_spec(d, h), _w_spec(p, h), _w_spec(h, d),
          _w_spec(d, d), _w_spec(1, d), _w_spec(1, d), _w_spec(1, d),
          _w_spec(d, h), _w_spec(d, h), _w_spec(h, d), _w_spec(1, d),
          _w_spec(1, d), _w_spec(d, hp), _w_spec(d, hp),
      ],
      out_specs=[_node_spec(d), _node_spec(hp), _node_spec(hp)],
      out_shape=[
          jax.ShapeDtypeStruct((n, d), jnp.float32),
          jax.ShapeDtypeStruct((n, hp), jnp.float32),
          jax.ShapeDtypeStruct((n, hp), jnp.float32),
      ],
  )(local, lg, paire, pme,
    w1s, w1n, w1p, W1b,
    Wgate_l, row(bgate_l), row(g1), row(b1),
    Wg, Wv, Wo, row(g2), row(b2),
    w2s, w2n)

  b2g = _gather32(b2t, idx)

  pair2 = pl.pallas_call(
      _tc2_body,
      grid=grid,
      in_specs=[
          _node_spec(hp), _edge_spec(hp), _edge_spec(p),
          _w_spec(p, hp), _w_spec(hp, p), _w_spec(p, p), _w_spec(1, p),
          _w_spec(1, p), _w_spec(1, p),
      ],
      out_specs=_edge_spec(p),
      out_shape=jax.ShapeDtypeStruct((n * k, p), jnp.float32),
  )(a2, b2g, paire,
    w2p, W2b, Wgate_p, row(bgate_p), row(g3), row(b3))

  return (local2, pair2.reshape(n, k, p))


# speedup vs baseline: 1.7264x; 1.7264x over previous
import functools

import jax
import jax.numpy as jnp
from jax import lax
from jax.experimental import pallas as pl
from jax.experimental.pallas import tpu as pltpu
from jax.experimental.pallas import tpu_sc as plsc

_N, _K, _D, _P = 10000, 16, 128, 16
_H = _D * 4
_HP = _P * 2
_BN = 200


def _make_sc_gather(n_idx, d, chunk):
  info = plsc.get_sparse_core_info()
  nw = info.num_cores * info.num_subcores
  b_per_w = n_idx // nw
  assert b_per_w * nw == n_idx and b_per_w % chunk == 0 and chunk % 8 == 0
  nchunks = b_per_w // chunk
  mesh = plsc.VectorSubcoreMesh(core_axis_name="c", subcore_axis_name="s")

  @functools.partial(
      pl.kernel,
      out_type=jax.ShapeDtypeStruct((n_idx, d), jnp.float32),
      mesh=mesh,
      scratch_types=[
          pltpu.VMEM((chunk,), jnp.int32),
          pltpu.VMEM((chunk, d), jnp.float32),
          pltpu.SemaphoreType.DMA,
      ],
  )
  def gather_kernel(table_hbm, idx_hbm, out_hbm, idx_v, rows_v, sem):
    wid = lax.axis_index("s") * info.num_cores + lax.axis_index("c")
    base = wid * b_per_w

    @pl.loop(0, nchunks)
    def _chunk(ci):
      off = pl.multiple_of(base + ci * chunk, 8)
      pltpu.sync_copy(idx_hbm.at[pl.ds(off, chunk)], idx_v)
      pltpu.async_copy(table_hbm.at[idx_v], rows_v, sem).wait()
      pltpu.sync_copy(rows_v, out_hbm.at[pl.ds(off, chunk)])

  return gather_kernel


@functools.lru_cache(maxsize=None)
def _sc_gather(n_idx, d, chunk):
  return _make_sc_gather(n_idx, d, chunk)


def _gather128(table, idx):
  return _sc_gather(_N * _K, _D, 200)(table, idx)


def _ln(x, g, b, eps=1e-5):
  mu = jnp.mean(x, axis=-1, keepdims=True)
  var = jnp.mean((x - mu) ** 2, axis=-1, keepdims=True)
  return (x - mu) / jnp.sqrt(var + eps) * g + b


def _tc1_body(local_ref, lg_ref, paire_ref, pme_ref,
              w1s_ref, w1n_ref, w1p_ref, w1b_ref,
              wgl_ref, bgl_ref, g1_ref, b1_ref,
              wg_ref, wv_ref, wo_ref, g2_ref, b2_ref,
              w2s_ref,
              out_local_ref, out_a2_ref):
  f32 = jnp.float32
  local = local_ref[...]
  a1 = jnp.dot(local, w1s_ref[...], preferred_element_type=f32)
  h = jnp.dot(lg_ref[...], w1n_ref[...], preferred_element_type=f32)
  h = h + jnp.dot(paire_ref[...], w1p_ref[...], preferred_element_type=f32)
  a1e = jnp.broadcast_to(a1[:, None, :], (_BN, _K, _H)).reshape(_BN * _K, _H)
  h = jnp.maximum(h + a1e, 0.0) * pme_ref[...]
  s = h.reshape(_BN, _K, _H).sum(axis=1)
  lu = jnp.dot(s, w1b_ref[...], preferred_element_type=f32) * (1.0 / _K)
  gate = jax.nn.sigmoid(
      jnp.dot(local, wgl_ref[...], preferred_element_type=f32) + bgl_ref[...])
  x = _ln(local + lu * gate, g1_ref[...], b1_ref[...])
  g = jnp.dot(x, wg_ref[...], preferred_element_type=f32)
  v = jnp.dot(x, wv_ref[...], preferred_element_type=f32)
  y = jnp.dot(jax.nn.silu(g) * v, wo_ref[...], preferred_element_type=f32)
  x2 = _ln(x + y, g2_ref[...], b2_ref[...])
  out_local_ref[...] = x2
  out_a2_ref[...] = jnp.dot(x2, w2s_ref[...], preferred_element_type=f32)


def _tc2_body(a2_ref, lg2_ref, paire_ref,
              w2n_ref, w2p_ref, w2b_ref, wgp_ref, bgp_ref, g3_ref, b3_ref,
              out_ref):
  f32 = jnp.float32
  a2 = a2_ref[...]
  a2e = jnp.broadcast_to(a2[:, None, :], (_BN, _K, _HP)).reshape(_BN * _K, _HP)
  paire = paire_ref[...]
  h = jnp.dot(paire, w2p_ref[...], preferred_element_type=f32)
  h = h + jnp.dot(lg2_ref[...], w2n_ref[...], preferred_element_type=f32)
  h = jnp.maximum(h + a2e, 0.0)
  pu = jnp.dot(h, w2b_ref[...], preferred_element_type=f32)
  gate = jax.nn.sigmoid(
      jnp.dot(paire, wgp_ref[...], preferred_element_type=f32) + bgp_ref[...])
  out_ref[...] = _ln(paire + pu * gate, g3_ref[...], b3_ref[...])


def _node_spec(d):
  return pl.BlockSpec((_BN, d), lambda i: (i, 0))


def _edge_spec(d):
  return pl.BlockSpec((_BN * _K, d), lambda i: (i, 0))


def _w_spec(r, c):
  return pl.BlockSpec((r, c), lambda i: (0, 0))


def kernel(local, pair, neighbours, mask, W1a, W1b, Wgate_l, bgate_l, g1, b1,
           Wg, Wv, Wo, g2, b2, W2a, W2b, Wgate_p, bgate_p, g3, b3):
  n, k = neighbours.shape
  d, p, h, hp = _D, _P, _H, _HP

  idx = jnp.where(neighbours < 0, 0, neighbours).reshape(n * k)
  pme = (mask[:, None] * (neighbours != -1).astype(jnp.float32)).reshape(n * k, 1)
  paire = pair.reshape(n * k, p)

  w1s, w1n, w1p = W1a[:d], W1a[d:2 * d], W1a[2 * d:]
  w2s, w2n, w2p = W2a[:d], W2a[d:2 * d], W2a[2 * d:]
  row = lambda a: a.reshape(1, -1)

  lg = _gather128(local, idx)

  grid = (n // _BN,)
  local2, a2 = pl.pallas_call(
      _tc1_body,
      grid=grid,
      in_specs=[
          _node_spec(d), _edge_spec(d), _edge_spec(p), _edge_spec(1),
          _w_spec(d, h), _w_spec(d, h), _w_spec(p, h), _w_spec(h, d),
          _w_spec(d, d), _w_spec(1, d), _w_spec(1, d), _w_spec(1, d),
          _w_spec(d, h), _w_spec(d, h), _w_spec(h, d), _w_spec(1, d),
          _w_spec(1, d), _w_spec(d, hp),
      ],
      out_specs=[_node_spec(d), _node_spec(hp)],
      out_shape=[
          jax.ShapeDtypeStruct((n, d), jnp.float32),
          jax.ShapeDtypeStruct((n, hp), jnp.float32),
      ],
  )(local, lg, paire, pme,
    w1s, w1n, w1p, W1b,
    Wgate_l, row(bgate_l), row(g1), row(b1),
    Wg, Wv, Wo, row(g2), row(b2),
    w2s)

  lg2 = _gather128(local2, idx)

  pair2 = pl.pallas_call(
      _tc2_body,
      grid=grid,
      in_specs=[
          _node_spec(hp), _edge_spec(d), _edge_spec(p),
          _w_spec(d, hp), _w_spec(p, hp), _w_spec(hp, p), _w_spec(p, p),
          _w_spec(1, p), _w_spec(1, p), _w_spec(1, p),
      ],
      out_specs=_edge_spec(p),
      out_shape=jax.ShapeDtypeStruct((n * k, p), jnp.float32),
  )(a2, lg2, paire,
    w2n, w2p, W2b, Wgate_p, row(bgate_p), row(g3), row(b3))

  return (local2, pair2.reshape(n, k, p))


# speedup vs baseline: 2.6344x; 1.5260x over previous
import functools

import jax
import jax.numpy as jnp
from jax import lax
from jax.experimental import pallas as pl
from jax.experimental.pallas import tpu as pltpu
from jax.experimental.pallas import tpu_sc as plsc

_N, _K, _D, _P = 10000, 16, 128, 16
_H = _D * 4
_HP = _P * 2
_BN = 400


def _make_sc_gather(n_idx, d, chunk):
  info = plsc.get_sparse_core_info()
  nw = info.num_cores * info.num_subcores
  b_per_w = n_idx // nw
  assert b_per_w * nw == n_idx and b_per_w % chunk == 0 and chunk % 8 == 0
  nchunks = b_per_w // chunk
  mesh = plsc.VectorSubcoreMesh(core_axis_name="c", subcore_axis_name="s")

  @functools.partial(
      pl.kernel,
      out_type=jax.ShapeDtypeStruct((n_idx, d), jnp.float32),
      mesh=mesh,
      scratch_types=[
          pltpu.VMEM((chunk,), jnp.int32),
          pltpu.VMEM((chunk, d), jnp.float32),
          pltpu.SemaphoreType.DMA,
      ],
  )
  def gather_kernel(table_hbm, idx_hbm, out_hbm, idx_v, rows_v, sem):
    wid = lax.axis_index("s") * info.num_cores + lax.axis_index("c")
    base = wid * b_per_w

    @pl.loop(0, nchunks)
    def _chunk(ci):
      off = pl.multiple_of(base + ci * chunk, 8)
      pltpu.sync_copy(idx_hbm.at[pl.ds(off, chunk)], idx_v)
      pltpu.async_copy(table_hbm.at[idx_v], rows_v, sem).wait()
      pltpu.sync_copy(rows_v, out_hbm.at[pl.ds(off, chunk)])

  return gather_kernel


@functools.lru_cache(maxsize=None)
def _sc_gather(n_idx, d, chunk):
  return _make_sc_gather(n_idx, d, chunk)


def _gather128(table, idx):
  return _sc_gather(_N * _K, _D, 200)(table, idx)


def _ln(x, g, b, eps=1e-5):
  mu = jnp.mean(x, axis=-1, keepdims=True)
  var = jnp.mean((x - mu) ** 2, axis=-1, keepdims=True)
  return (x - mu) / jnp.sqrt(var + eps) * g + b


def _tc1_body(local_ref, lg_ref, pair_ref, pm_ref,
              w1s_ref, w1n_ref, w1p_ref, w1b_ref,
              wgl_ref, bgl_ref, g1_ref, b1_ref,
              wg_ref, wv_ref, wo_ref, g2_ref, b2_ref,
              w2s_ref,
              out_local_ref, out_a2_ref):
  f32 = jnp.float32
  dot = lambda a, b: jnp.dot(a, b, preferred_element_type=f32)
  local = local_ref[...]
  pair2d = pair_ref[...]
  pm = pm_ref[...]
  a1 = dot(local, w1s_ref[...])
  w1n = w1n_ref[...]
  w1p = w1p_ref[...]
  acc = jnp.zeros((_BN, _H), f32)
  for k in range(_K):
    hk = dot(lg_ref[k], w1n) + dot(pair2d[:, k * _P:(k + 1) * _P], w1p) + a1
    acc = acc + jnp.maximum(hk, 0.0) * pm[:, k:k + 1]
  lu = dot(acc, w1b_ref[...]) * (1.0 / _K)
  gate = jax.nn.sigmoid(dot(local, wgl_ref[...]) + bgl_ref[...])
  x = _ln(local + lu * gate, g1_ref[...], b1_ref[...])
  g = dot(x, wg_ref[...])
  v = dot(x, wv_ref[...])
  y = dot(jax.nn.silu(g) * v, wo_ref[...])
  x2 = _ln(x + y, g2_ref[...], b2_ref[...])
  out_local_ref[...] = x2
  out_a2_ref[...] = dot(x2, w2s_ref[...])


def _tc2_body(a2_ref, lg2_ref, pair_ref,
              w2n_ref, bdw2p_ref, bdw2b_ref, bdwgp_ref,
              bgpt_ref, g3t_ref, b3t_ref, mavg_ref,
              out_ref):
  f32 = jnp.float32
  dot = lambda a, b: jnp.dot(a, b, preferred_element_type=f32)
  pair2d = pair_ref[...]
  w2n = w2n_ref[...]
  b2g = jnp.concatenate([dot(lg2_ref[k], w2n) for k in range(_K)], axis=-1)
  a2t = jnp.concatenate([a2_ref[...]] * _K, axis=-1)
  h = jnp.maximum(dot(pair2d, bdw2p_ref[...]) + b2g + a2t, 0.0)
  pu = dot(h, bdw2b_ref[...])
  gate = jax.nn.sigmoid(dot(pair2d, bdwgp_ref[...]) + bgpt_ref[...])
  x = pair2d + pu * gate
  mavg = mavg_ref[...]
  mu = dot(x, mavg)
  xc = x - mu
  var = dot(xc * xc, mavg)
  out_ref[...] = xc * lax.rsqrt(var + 1e-5) * g3t_ref[...] + b3t_ref[...]


def _node_spec(d):
  return pl.BlockSpec((_BN, d), lambda i: (i, 0))


def _k_spec(d):
  return pl.BlockSpec((_K, _BN, d), lambda i: (0, i, 0))


def _w_spec(r, c):
  return pl.BlockSpec((r, c), lambda i: (0, 0))


def kernel(local, pair, neighbours, mask, W1a, W1b, Wgate_l, bgate_l, g1, b1,
           Wg, Wv, Wo, g2, b2, W2a, W2b, Wgate_p, bgate_p, g3, b3):
  n, kk = neighbours.shape
  d, p, h, hp = _D, _P, _H, _HP
  f32 = jnp.float32

  idx_t = neighbours.T.reshape(n * kk)
  pm = mask[:, None] * (neighbours != -1).astype(f32)
  pair2d = pair.reshape(n, kk * p)

  w1s, w1n, w1p = W1a[:d], W1a[d:2 * d], W1a[2 * d:]
  w2s, w2n, w2p = W2a[:d], W2a[d:2 * d], W2a[2 * d:]
  row = lambda a: a.reshape(1, -1)
  eye = jnp.eye(kk, dtype=f32)
  bd = lambda w: jnp.kron(eye, w)
  tile = lambda a: row(jnp.tile(a, kk))

  lg = _gather128(local, idx_t).reshape(kk, n, d)

  grid = (n // _BN,)
  local2, a2 = pl.pallas_call(
      _tc1_body,
      grid=grid,
      in_specs=[
          _node_spec(d), _k_spec(d), _node_spec(kk * p), _node_spec(kk),
          _w_spec(d, h), _w_spec(d, h), _w_spec(p, h), _w_spec(h, d),
          _w_spec(d, d), _w_spec(1, d), _w_spec(1, d), _w_spec(1, d),
          _w_spec(d, h), _w_spec(d, h), _w_spec(h, d), _w_spec(1, d),
          _w_spec(1, d), _w_spec(d, hp),
      ],
      out_specs=[_node_spec(d), _node_spec(hp)],
      out_shape=[
          jax.ShapeDtypeStruct((n, d), f32),
          jax.ShapeDtypeStruct((n, hp), f32),
      ],
  )(local, lg, pair2d, pm,
    w1s, w1n, w1p, W1b,
    Wgate_l, row(bgate_l), row(g1), row(b1),
    Wg, Wv, Wo, row(g2), row(b2),
    w2s)

  lg2 = _gather128(local2, idx_t).reshape(kk, n, d)

  pair_out = pl.pallas_call(
      _tc2_body,
      grid=grid,
      in_specs=[
          _node_spec(hp), _k_spec(d), _node_spec(kk * p),
          _w_spec(d, hp), _w_spec(kk * p, kk * hp), _w_spec(kk * hp, kk * p),
          _w_spec(kk * p, kk * p),
          _w_spec(1, kk * p), _w_spec(1, kk * p), _w_spec(1, kk * p),
          _w_spec(kk * p, kk * p),
      ],
      out_specs=_node_spec(kk * p),
      out_shape=jax.ShapeDtypeStruct((n, kk * p), f32),
  )(a2, lg2, pair2d,
    w2n, bd(w2p), bd(W2b), bd(Wgate_p),
    tile(bgate_p), tile(g3), tile(b3),
    bd(jnp.full((kk, kk), 1.0 / kk, f32)))

  return (local2, pair_out.reshape(n, kk, p))
